# Initial kernel scaffold; baseline (speedup 1.0000x reference)
import jax
import jax.numpy as jnp
from jax.experimental import pallas as pl


def kernel(hidden_states, Wq, bq, Wk, bk, Wv, bv, We, be, Wg, bg, Wo, bo):
    raise NotImplementedError("write your pallas kernel here")


# speedup vs baseline: 1.0840x; 1.0840x over previous
import functools
import math

import jax
import jax.numpy as jnp
from jax import lax
from jax.experimental import pallas as pl
from jax.experimental.pallas import tpu as pltpu

B, S, D = 2, 2048, 1024
H = 16
E = 8
DH = D // H

ROW_BLK = 512
SQ = 512
RB = 256
SB = 512

_CONTRACT_LAST = (((1,), (1,)), ((), ()))
_CONTRACT_STD = (((1,), (0,)), ((), ()))


def _qkv_body(x_ref, wq_ref, bq_ref, wk_ref, bk_ref, wv_ref, bv_ref,
              q_ref, k_ref, v_ref):
    x = x_ref[...]
    q_ref[...] = lax.dot_general(x, wq_ref[...], _CONTRACT_LAST,
                                 preferred_element_type=jnp.float32) + bq_ref[...]
    k_ref[...] = lax.dot_general(x, wk_ref[...], _CONTRACT_LAST,
                                 preferred_element_type=jnp.float32) + bk_ref[...]
    v_ref[...] = lax.dot_general(x, wv_ref[...], _CONTRACT_LAST,
                                 preferred_element_type=jnp.float32) + bv_ref[...]


def _attn_body(q_ref, k_ref, v_ref, ctx_ref, mean_ref):
    qi = pl.program_id(2)

    q = q_ref[0, 0]
    k = k_ref[0, 0]
    v = v_ref[0, 0]

    scores = lax.dot_general(q, k, _CONTRACT_LAST,
                             preferred_element_type=jnp.float32)
    scores = scores * (1.0 / math.sqrt(DH))
    m = jnp.max(scores, axis=1, keepdims=True)
    p = jnp.exp(scores - m)
    l = jnp.sum(p, axis=1, keepdims=True)
    ctx = lax.dot_general(p, v, _CONTRACT_STD,
                          preferred_element_type=jnp.float32) / l
    ctx_ref[0, 0] = ctx

    @pl.when(qi == 0)
    def _():
        mean_ref[...] = jnp.zeros_like(mean_ref)

    part = jnp.sum(ctx, axis=0, keepdims=True) * (1.0 / S)
    mean_ref[0, 0, 0:1, :] += part


def _gate_body(g_ref, wg_ref, bg_ref, be_ref, w_ref, bc_ref):
    g = g_ref[...]
    logits = lax.dot_general(g, wg_ref[...], _CONTRACT_LAST,
                             preferred_element_type=jnp.float32) + bg_ref[...]
    m = jnp.max(logits, axis=1, keepdims=True)
    pexp = jnp.exp(logits - m)
    probs = pexp / jnp.sum(pexp, axis=1, keepdims=True)

    eidx = lax.broadcasted_iota(jnp.int32, (B, E), 1)
    v1 = jnp.max(probs, axis=1, keepdims=True)
    i1 = jnp.min(jnp.where(probs >= v1, eidx, E), axis=1, keepdims=True)
    mask1 = eidx == i1
    p2 = jnp.where(mask1, -1.0, probs)
    v2 = jnp.max(p2, axis=1, keepdims=True)
    i2 = jnp.min(jnp.where(p2 >= v2, eidx, E), axis=1, keepdims=True)
    mask2 = eidx == i2

    w = jnp.where(mask1, v1, 0.0) + jnp.where(mask2, v2, 0.0)
    w_ref[...] = w
    bc_ref[...] = lax.dot_general(w, be_ref[...], _CONTRACT_STD,
                                  preferred_element_type=jnp.float32)


def _combine_body(w_ref, we_ref, wc_ref):
    we = we_ref[...]
    for b in range(B):
        acc = w_ref[b, 0] * we[0]
        for e in range(1, E):
            acc = acc + w_ref[b, e] * we[e]
        wc_ref[b] = acc


def _moe_out_body(ctx_ref, wc_ref, bc_ref, wo_ref, bo_ref, out_ref):
    ctx = ctx_ref[0]
    moe = lax.dot_general(ctx, wc_ref[0], _CONTRACT_LAST,
                          preferred_element_type=jnp.float32) + bc_ref[0]
    out = lax.dot_general(moe, wo_ref[...], _CONTRACT_LAST,
                          preferred_element_type=jnp.float32) + bo_ref[...]
    out_ref[0] = out


def kernel(hidden_states, Wq, bq, Wk, bk, Wv, bv, We, be, Wg, bg, Wo, bo):
    x2d = hidden_states.reshape(B * S, D)
    bq2 = bq.reshape(1, D)
    bk2 = bk.reshape(1, D)
    bv2 = bv.reshape(1, D)
    bg2 = bg.reshape(1, E)
    bo2 = bo.reshape(1, D)

    n_row = (B * S) // ROW_BLK
    q2d, k2d, v2d = pl.pallas_call(
        _qkv_body,
        grid=(n_row,),
        in_specs=[
            pl.BlockSpec((ROW_BLK, D), lambda i: (i, 0)),
            pl.BlockSpec((D, D), lambda i: (0, 0)),
            pl.BlockSpec((1, D), lambda i: (0, 0)),
            pl.BlockSpec((D, D), lambda i: (0, 0)),
            pl.BlockSpec((1, D), lambda i: (0, 0)),
            pl.BlockSpec((D, D), lambda i: (0, 0)),
            pl.BlockSpec((1, D), lambda i: (0, 0)),
        ],
        out_specs=[
            pl.BlockSpec((ROW_BLK, D), lambda i: (i, 0)),
            pl.BlockSpec((ROW_BLK, D), lambda i: (i, 0)),
            pl.BlockSpec((ROW_BLK, D), lambda i: (i, 0)),
        ],
        out_shape=[jax.ShapeDtypeStruct((B * S, D), jnp.float32)] * 3,
    )(x2d, Wq, bq2, Wk, bk2, Wv, bv2)

    q = q2d.reshape(B, S, H, DH).transpose(0, 2, 1, 3)
    k = k2d.reshape(B, S, H, DH).transpose(0, 2, 1, 3)
    v = v2d.reshape(B, S, H, DH).transpose(0, 2, 1, 3)

    nq = S // SQ
    ctx4, means = pl.pallas_call(
        _attn_body,
        grid=(B, H, nq),
        in_specs=[
            pl.BlockSpec((1, 1, SQ, DH), lambda b, h, qi: (b, h, qi, 0)),
            pl.BlockSpec((1, 1, S, DH), lambda b, h, qi: (b, h, 0, 0)),
            pl.BlockSpec((1, 1, S, DH), lambda b, h, qi: (b, h, 0, 0)),
        ],
        out_specs=[
            pl.BlockSpec((1, 1, SQ, DH), lambda b, h, qi: (b, h, qi, 0)),
            pl.BlockSpec((1, 1, 8, DH), lambda b, h, qi: (b, h, 0, 0)),
        ],
        out_shape=[
            jax.ShapeDtypeStruct((B, H, S, DH), jnp.float32),
            jax.ShapeDtypeStruct((B, H, 8, DH), jnp.float32),
        ],
    )(q, k, v)

    ctx = ctx4.transpose(0, 2, 1, 3).reshape(B, S, D)
    gate_input = means[:, :, 0, :].reshape(B, D)

    w, bc = pl.pallas_call(
        _gate_body,
        grid=(1,),
        in_specs=[
            pl.BlockSpec((B, D), lambda i: (0, 0)),
            pl.BlockSpec((E, D), lambda i: (0, 0)),
            pl.BlockSpec((1, E), lambda i: (0, 0)),
            pl.BlockSpec((E, D), lambda i: (0, 0)),
        ],
        out_specs=[
            pl.BlockSpec((B, E), lambda i: (0, 0)),
            pl.BlockSpec((B, D), lambda i: (0, 0)),
        ],
        out_shape=[
            jax.ShapeDtypeStruct((B, E), jnp.float32),
            jax.ShapeDtypeStruct((B, D), jnp.float32),
        ],
    )(gate_input, Wg, bg2, be)

    n_rb = D // RB
    wc = pl.pallas_call(
        _combine_body,
        grid=(n_rb,),
        in_specs=[
            pl.BlockSpec(memory_space=pltpu.SMEM),
            pl.BlockSpec((E, RB, D), lambda i: (0, i, 0)),
        ],
        out_specs=pl.BlockSpec((B, RB, D), lambda i: (0, i, 0)),
        out_shape=jax.ShapeDtypeStruct((B, D, D), jnp.float32),
    )(w, We)

    bc3 = bc.reshape(B, 1, D)
    ns = S // SB
    out = pl.pallas_call(
        _moe_out_body,
        grid=(B, ns),
        in_specs=[
            pl.BlockSpec((1, SB, D), lambda b, si: (b, si, 0)),
            pl.BlockSpec((1, D, D), lambda b, si: (b, 0, 0)),
            pl.BlockSpec((1, 1, D), lambda b, si: (b, 0, 0)),
            pl.BlockSpec((D, D), lambda b, si: (0, 0)),
            pl.BlockSpec((1, D), lambda b, si: (0, 0)),
        ],
        out_specs=pl.BlockSpec((1, SB, D), lambda b, si: (b, si, 0)),
        out_shape=jax.ShapeDtypeStruct((B, S, D), jnp.float32),
    )(ctx, wc, bc3, Wo, bo2)

    return out


# speedup vs baseline: 1.1347x; 1.0468x over previous
import functools
import math

import jax
import jax.numpy as jnp
from jax import lax
from jax.experimental import pallas as pl
from jax.experimental.pallas import tpu as pltpu

B, S, D = 2, 2048, 1024
H = 16
E = 8
DH = D // H

ROW_BLK = 512
SQ = 512
RB = 256
SB = 512

_CONTRACT_LAST = (((1,), (1,)), ((), ()))
_CONTRACT_STD = (((1,), (0,)), ((), ()))


def _qkv_body(x_ref, wq_ref, bq_ref, wk_ref, bk_ref, wv_ref, bv_ref,
              q_ref, k_ref, v_ref):
    x = x_ref[...].astype(jnp.bfloat16)
    q = lax.dot_general(x, wq_ref[...], _CONTRACT_LAST,
                        preferred_element_type=jnp.float32) + bq_ref[...]
    k = lax.dot_general(x, wk_ref[...], _CONTRACT_LAST,
                        preferred_element_type=jnp.float32) + bk_ref[...]
    v = lax.dot_general(x, wv_ref[...], _CONTRACT_LAST,
                        preferred_element_type=jnp.float32) + bv_ref[...]
    q_ref[...] = q.astype(jnp.bfloat16)
    k_ref[...] = k.astype(jnp.bfloat16)
    v_ref[...] = v.astype(jnp.bfloat16)


def _attn_body(q_ref, k_ref, v_ref, ctx_ref, mean_ref):
    qi = pl.program_id(2)

    q = q_ref[0, 0]
    k = k_ref[0, 0]
    v = v_ref[0, 0]

    scores = lax.dot_general(q, k, _CONTRACT_LAST,
                             preferred_element_type=jnp.float32)
    m = jnp.max(scores, axis=1, keepdims=True)
    p = jnp.exp(scores - m)
    l = jnp.sum(p, axis=1, keepdims=True)
    ctx = lax.dot_general(p.astype(jnp.bfloat16), v, _CONTRACT_STD,
                          preferred_element_type=jnp.float32) / l
    ctx_ref[0, 0] = ctx

    @pl.when(qi == 0)
    def _():
        mean_ref[...] = jnp.zeros_like(mean_ref)

    part = jnp.sum(ctx, axis=0, keepdims=True) * (1.0 / S)
    mean_ref[0, 0, 0:1, :] += part


def _gate_body(g_ref, wg_ref, bg_ref, be_ref, w_ref, bc_ref):
    g = g_ref[...]
    logits = lax.dot_general(g, wg_ref[...], _CONTRACT_LAST,
                             preferred_element_type=jnp.float32) + bg_ref[...]
    m = jnp.max(logits, axis=1, keepdims=True)
    pexp = jnp.exp(logits - m)
    probs = pexp / jnp.sum(pexp, axis=1, keepdims=True)

    eidx = lax.broadcasted_iota(jnp.int32, (B, E), 1)
    v1 = jnp.max(probs, axis=1, keepdims=True)
    i1 = jnp.min(jnp.where(probs >= v1, eidx, E), axis=1, keepdims=True)
    mask1 = eidx == i1
    p2 = jnp.where(mask1, -1.0, probs)
    v2 = jnp.max(p2, axis=1, keepdims=True)
    i2 = jnp.min(jnp.where(p2 >= v2, eidx, E), axis=1, keepdims=True)
    mask2 = eidx == i2

    w = jnp.where(mask1, v1, 0.0) + jnp.where(mask2, v2, 0.0)
    w_ref[...] = w
    bc_ref[...] = lax.dot_general(w, be_ref[...], _CONTRACT_STD,
                                  preferred_element_type=jnp.float32)


def _combine_body(w_ref, we_ref, wc_ref):
    we = we_ref[...].astype(jnp.float32)
    for b in range(B):
        acc = w_ref[b, 0] * we[0]
        for e in range(1, E):
            acc = acc + w_ref[b, e] * we[e]
        wc_ref[b] = acc.astype(jnp.bfloat16)


def _moe_out_body(ctx_ref, wc_ref, bc_ref, wo_ref, bo_ref, out_ref):
    ctx = ctx_ref[0].astype(jnp.bfloat16)
    moe = lax.dot_general(ctx, wc_ref[0], _CONTRACT_LAST,
                          preferred_element_type=jnp.float32) + bc_ref[0]
    out = lax.dot_general(moe.astype(jnp.bfloat16), wo_ref[...], _CONTRACT_LAST,
                          preferred_element_type=jnp.float32) + bo_ref[...]
    out_ref[0] = out


def kernel(hidden_states, Wq, bq, Wk, bk, Wv, bv, We, be, Wg, bg, Wo, bo):
    x2d = hidden_states.reshape(B * S, D)
    scale = 1.0 / math.sqrt(DH)
    wq16 = (Wq * scale).astype(jnp.bfloat16)
    wk16 = Wk.astype(jnp.bfloat16)
    wv16 = Wv.astype(jnp.bfloat16)
    we16 = We.astype(jnp.bfloat16)
    wo16 = Wo.astype(jnp.bfloat16)
    bq2 = (bq * scale).reshape(1, D)
    bk2 = bk.reshape(1, D)
    bv2 = bv.reshape(1, D)
    bg2 = bg.reshape(1, E)
    bo2 = bo.reshape(1, D)

    n_row = (B * S) // ROW_BLK
    q2d, k2d, v2d = pl.pallas_call(
        _qkv_body,
        grid=(n_row,),
        in_specs=[
            pl.BlockSpec((ROW_BLK, D), lambda i: (i, 0)),
            pl.BlockSpec((D, D), lambda i: (0, 0)),
            pl.BlockSpec((1, D), lambda i: (0, 0)),
            pl.BlockSpec((D, D), lambda i: (0, 0)),
            pl.BlockSpec((1, D), lambda i: (0, 0)),
            pl.BlockSpec((D, D), lambda i: (0, 0)),
            pl.BlockSpec((1, D), lambda i: (0, 0)),
        ],
        out_specs=[
            pl.BlockSpec((ROW_BLK, D), lambda i: (i, 0)),
            pl.BlockSpec((ROW_BLK, D), lambda i: (i, 0)),
            pl.BlockSpec((ROW_BLK, D), lambda i: (i, 0)),
        ],
        out_shape=[jax.ShapeDtypeStruct((B * S, D), jnp.bfloat16)] * 3,
    )(x2d, wq16, bq2, wk16, bk2, wv16, bv2)

    q = q2d.reshape(B, S, H, DH).transpose(0, 2, 1, 3)
    k = k2d.reshape(B, S, H, DH).transpose(0, 2, 1, 3)
    v = v2d.reshape(B, S, H, DH).transpose(0, 2, 1, 3)

    nq = S // SQ
    ctx4, means = pl.pallas_call(
        _attn_body,
        grid=(B, H, nq),
        in_specs=[
            pl.BlockSpec((1, 1, SQ, DH), lambda b, h, qi: (b, h, qi, 0)),
            pl.BlockSpec((1, 1, S, DH), lambda b, h, qi: (b, h, 0, 0)),
            pl.BlockSpec((1, 1, S, DH), lambda b, h, qi: (b, h, 0, 0)),
        ],
        out_specs=[
            pl.BlockSpec((1, 1, SQ, DH), lambda b, h, qi: (b, h, qi, 0)),
            pl.BlockSpec((1, 1, 8, DH), lambda b, h, qi: (b, h, 0, 0)),
        ],
        out_shape=[
            jax.ShapeDtypeStruct((B, H, S, DH), jnp.float32),
            jax.ShapeDtypeStruct((B, H, 8, DH), jnp.float32),
        ],
    )(q, k, v)

    ctx = ctx4.transpose(0, 2, 1, 3).reshape(B, S, D)
    gate_input = means[:, :, 0, :].reshape(B, D)

    w, bc = pl.pallas_call(
        _gate_body,
        grid=(1,),
        in_specs=[
            pl.BlockSpec((B, D), lambda i: (0, 0)),
            pl.BlockSpec((E, D), lambda i: (0, 0)),
            pl.BlockSpec((1, E), lambda i: (0, 0)),
            pl.BlockSpec((E, D), lambda i: (0, 0)),
        ],
        out_specs=[
            pl.BlockSpec((B, E), lambda i: (0, 0)),
            pl.BlockSpec((B, D), lambda i: (0, 0)),
        ],
        out_shape=[
            jax.ShapeDtypeStruct((B, E), jnp.float32),
            jax.ShapeDtypeStruct((B, D), jnp.float32),
        ],
    )(gate_input, Wg, bg2, be)

    n_rb = D // RB
    wc = pl.pallas_call(
        _combine_body,
        grid=(n_rb,),
        in_specs=[
            pl.BlockSpec(memory_space=pltpu.SMEM),
            pl.BlockSpec((E, RB, D), lambda i: (0, i, 0)),
        ],
        out_specs=pl.BlockSpec((B, RB, D), lambda i: (0, i, 0)),
        out_shape=jax.ShapeDtypeStruct((B, D, D), jnp.bfloat16),
    )(w, we16)

    bc3 = bc.reshape(B, 1, D)
    ns = S // SB
    out = pl.pallas_call(
        _moe_out_body,
        grid=(B, ns),
        in_specs=[
            pl.BlockSpec((1, SB, D), lambda b, si: (b, si, 0)),
            pl.BlockSpec((1, D, D), lambda b, si: (b, 0, 0)),
            pl.BlockSpec((1, 1, D), lambda b, si: (b, 0, 0)),
            pl.BlockSpec((D, D), lambda b, si: (0, 0)),
            pl.BlockSpec((1, D), lambda b, si: (0, 0)),
        ],
        out_specs=pl.BlockSpec((1, SB, D), lambda b, si: (b, si, 0)),
        out_shape=jax.ShapeDtypeStruct((B, S, D), jnp.float32),
    )(ctx, wc, bc3, wo16, bo2)

    return out


# speedup vs baseline: 1.9121x; 1.6851x over previous
import functools
import math

import jax
import jax.numpy as jnp
from jax import lax
from jax.experimental import pallas as pl
from jax.experimental.pallas import tpu as pltpu

B, S, D = 2, 2048, 1024
H = 16
E = 8
DH = D // H

SBLK = 512
SQ = 512
HQ = SQ // 2
RB = 256
SB = 512

_CONTRACT_LAST = (((1,), (1,)), ((), ()))
_CONTRACT_STD = (((1,), (0,)), ((), ()))


def _qkv_body(x_ref, wq_ref, bq_ref, wk_ref, bk_ref, wv_ref, bv_ref,
              q_ref, k_ref, v_ref):
    x = x_ref[0].astype(jnp.bfloat16)
    q = (lax.dot_general(x, wq_ref[...], _CONTRACT_LAST,
                         preferred_element_type=jnp.float32)
         + bq_ref[...]).astype(jnp.bfloat16)
    k = (lax.dot_general(x, wk_ref[...], _CONTRACT_LAST,
                         preferred_element_type=jnp.float32)
         + bk_ref[...]).astype(jnp.bfloat16)
    v = (lax.dot_general(x, wv_ref[...], _CONTRACT_LAST,
                         preferred_element_type=jnp.float32)
         + bv_ref[...]).astype(jnp.bfloat16)
    for h in range(H):
        sl = slice(h * DH, (h + 1) * DH)
        q_ref[0, h] = q[:, sl]
        k_ref[0, h] = k[:, sl]
        v_ref[0, h] = v[:, sl]


def _attn_body(q_ref, k_ref, v_ref, ctx_ref, mean_ref):
    qi = pl.program_id(2)

    k = k_ref[0, 0]
    v = v_ref[0, 0]

    part = jnp.zeros((1, DH), jnp.float32)
    for half in range(2):
        q = q_ref[0, 0, half * HQ:(half + 1) * HQ]
        scores = lax.dot_general(q, k, _CONTRACT_LAST,
                                 preferred_element_type=jnp.float32)
        m = jnp.max(scores, axis=1, keepdims=True)
        p = jnp.exp(scores - m)
        l = jnp.sum(p, axis=1, keepdims=True)
        ctx = lax.dot_general(p.astype(jnp.bfloat16), v, _CONTRACT_STD,
                              preferred_element_type=jnp.float32) / l
        ctx_ref[0, 0, half * HQ:(half + 1) * HQ] = ctx.astype(jnp.bfloat16)
        part = part + jnp.sum(ctx, axis=0, keepdims=True)

    @pl.when(qi == 0)
    def _():
        mean_ref[...] = jnp.zeros_like(mean_ref)

    mean_ref[0, 0, 0:1, :] += part * (1.0 / S)


def _gate_body(g_ref, wg_ref, bg_ref, be_ref, w_ref, bc_ref):
    g = g_ref[...]
    logits = lax.dot_general(g, wg_ref[...], _CONTRACT_LAST,
                             preferred_element_type=jnp.float32) + bg_ref[...]
    m = jnp.max(logits, axis=1, keepdims=True)
    pexp = jnp.exp(logits - m)
    probs = pexp / jnp.sum(pexp, axis=1, keepdims=True)

    eidx = lax.broadcasted_iota(jnp.int32, (B, E), 1)
    v1 = jnp.max(probs, axis=1, keepdims=True)
    i1 = jnp.min(jnp.where(probs >= v1, eidx, E), axis=1, keepdims=True)
    mask1 = eidx == i1
    p2 = jnp.where(mask1, -1.0, probs)
    v2 = jnp.max(p2, axis=1, keepdims=True)
    i2 = jnp.min(jnp.where(p2 >= v2, eidx, E), axis=1, keepdims=True)
    mask2 = eidx == i2

    w = jnp.where(mask1, v1, 0.0) + jnp.where(mask2, v2, 0.0)
    w_ref[...] = w
    bc_ref[...] = lax.dot_general(w, be_ref[...], _CONTRACT_STD,
                                  preferred_element_type=jnp.float32)


def _combine_body(w_ref, we_ref, wc_ref):
    we = we_ref[...].astype(jnp.float32)
    for b in range(B):
        acc = w_ref[b, 0] * we[0]
        for e in range(1, E):
            acc = acc + w_ref[b, e] * we[e]
        wc_ref[b] = acc.astype(jnp.bfloat16)


def _moe_out_body(ctx_ref, wc_ref, bc_ref, wo_ref, bo_ref, out_ref):
    ctx = jnp.concatenate([ctx_ref[0, h] for h in range(H)], axis=1)
    moe = lax.dot_general(ctx, wc_ref[0], _CONTRACT_LAST,
                          preferred_element_type=jnp.float32) + bc_ref[0]
    out = lax.dot_general(moe.astype(jnp.bfloat16), wo_ref[...], _CONTRACT_LAST,
                          preferred_element_type=jnp.float32) + bo_ref[...]
    out_ref[0] = out


def kernel(hidden_states, Wq, bq, Wk, bk, Wv, bv, We, be, Wg, bg, Wo, bo):
    scale = 1.0 / math.sqrt(DH)
    wq16 = (Wq * scale).astype(jnp.bfloat16)
    wk16 = Wk.astype(jnp.bfloat16)
    wv16 = Wv.astype(jnp.bfloat16)
    we16 = We.astype(jnp.bfloat16)
    wo16 = Wo.astype(jnp.bfloat16)
    bq2 = (bq * scale).reshape(1, D)
    bk2 = bk.reshape(1, D)
    bv2 = bv.reshape(1, D)
    bg2 = bg.reshape(1, E)
    bo2 = bo.reshape(1, D)

    n_sb = S // SBLK
    qkv_struct = jax.ShapeDtypeStruct((B, H, S, DH), jnp.bfloat16)
    q, k, v = pl.pallas_call(
        _qkv_body,
        grid=(B, n_sb),
        in_specs=[
            pl.BlockSpec((1, SBLK, D), lambda b, si: (b, si, 0)),
            pl.BlockSpec((D, D), lambda b, si: (0, 0)),
            pl.BlockSpec((1, D), lambda b, si: (0, 0)),
            pl.BlockSpec((D, D), lambda b, si: (0, 0)),
            pl.BlockSpec((1, D), lambda b, si: (0, 0)),
            pl.BlockSpec((D, D), lambda b, si: (0, 0)),
            pl.BlockSpec((1, D), lambda b, si: (0, 0)),
        ],
        out_specs=[
            pl.BlockSpec((1, H, SBLK, DH), lambda b, si: (b, 0, si, 0)),
            pl.BlockSpec((1, H, SBLK, DH), lambda b, si: (b, 0, si, 0)),
            pl.BlockSpec((1, H, SBLK, DH), lambda b, si: (b, 0, si, 0)),
        ],
        out_shape=[qkv_struct] * 3,
    )(hidden_states, wq16, bq2, wk16, bk2, wv16, bv2)

    nq = S // SQ
    ctx4, means = pl.pallas_call(
        _attn_body,
        grid=(B, H, nq),
        in_specs=[
            pl.BlockSpec((1, 1, SQ, DH), lambda b, h, qi: (b, h, qi, 0)),
            pl.BlockSpec((1, 1, S, DH), lambda b, h, qi: (b, h, 0, 0)),
            pl.BlockSpec((1, 1, S, DH), lambda b, h, qi: (b, h, 0, 0)),
        ],
        out_specs=[
            pl.BlockSpec((1, 1, SQ, DH), lambda b, h, qi: (b, h, qi, 0)),
            pl.BlockSpec((1, 1, 8, DH), lambda b, h, qi: (b, h, 0, 0)),
        ],
        out_shape=[
            jax.ShapeDtypeStruct((B, H, S, DH), jnp.bfloat16),
            jax.ShapeDtypeStruct((B, H, 8, DH), jnp.float32),
        ],
    )(q, k, v)

    gate_input = means[:, :, 0, :].reshape(B, D)

    w, bc = pl.pallas_call(
        _gate_body,
        grid=(1,),
        in_specs=[
            pl.BlockSpec((B, D), lambda i: (0, 0)),
            pl.BlockSpec((E, D), lambda i: (0, 0)),
            pl.BlockSpec((1, E), lambda i: (0, 0)),
            pl.BlockSpec((E, D), lambda i: (0, 0)),
        ],
        out_specs=[
            pl.BlockSpec((B, E), lambda i: (0, 0)),
            pl.BlockSpec((B, D), lambda i: (0, 0)),
        ],
        out_shape=[
            jax.ShapeDtypeStruct((B, E), jnp.float32),
            jax.ShapeDtypeStruct((B, D), jnp.float32),
        ],
    )(gate_input, Wg, bg2, be)

    n_rb = D // RB
    wc = pl.pallas_call(
        _combine_body,
        grid=(n_rb,),
        in_specs=[
            pl.BlockSpec(memory_space=pltpu.SMEM),
            pl.BlockSpec((E, RB, D), lambda i: (0, i, 0)),
        ],
        out_specs=pl.BlockSpec((B, RB, D), lambda i: (0, i, 0)),
        out_shape=jax.ShapeDtypeStruct((B, D, D), jnp.bfloat16),
    )(w, we16)

    bc3 = bc.reshape(B, 1, D)
    ns = S // SB
    out = pl.pallas_call(
        _moe_out_body,
        grid=(B, ns),
        in_specs=[
            pl.BlockSpec((1, H, SB, DH), lambda b, si: (b, 0, si, 0)),
            pl.BlockSpec((1, D, D), lambda b, si: (b, 0, 0)),
            pl.BlockSpec((1, 1, D), lambda b, si: (b, 0, 0)),
            pl.BlockSpec((D, D), lambda b, si: (0, 0)),
            pl.BlockSpec((1, D), lambda b, si: (0, 0)),
        ],
        out_specs=pl.BlockSpec((1, SB, D), lambda b, si: (b, si, 0)),
        out_shape=jax.ShapeDtypeStruct((B, S, D), jnp.float32),
    )(ctx4, wc, bc3, wo16, bo2)

    return out


# speedup vs baseline: 2.0498x; 1.0720x over previous
import functools
import math

import jax
import jax.numpy as jnp
from jax import lax
from jax.experimental import pallas as pl
from jax.experimental.pallas import tpu as pltpu

B, S, D = 2, 2048, 1024
H = 16
E = 8
DH = D // H

SBLK = 512
SQ = 1024
NCH = 4
HQ = SQ // NCH
RB = 256
SB = 512

_CONTRACT_LAST = (((1,), (1,)), ((), ()))
_CONTRACT_STD = (((1,), (0,)), ((), ()))


def _qkv_body(x_ref, wq_ref, bq_ref, wk_ref, bk_ref, wv_ref, bv_ref,
              q_ref, k_ref, v_ref):
    x = x_ref[0].astype(jnp.bfloat16)
    q = (lax.dot_general(x, wq_ref[...], _CONTRACT_LAST,
                         preferred_element_type=jnp.float32)
         + bq_ref[...]).astype(jnp.bfloat16)
    k = (lax.dot_general(x, wk_ref[...], _CONTRACT_LAST,
                         preferred_element_type=jnp.float32)
         + bk_ref[...]).astype(jnp.bfloat16)
    v = (lax.dot_general(x, wv_ref[...], _CONTRACT_LAST,
                         preferred_element_type=jnp.float32)
         + bv_ref[...]).astype(jnp.bfloat16)
    for h in range(H):
        sl = slice(h * DH, (h + 1) * DH)
        q_ref[0, h] = q[:, sl]
        k_ref[0, h] = k[:, sl]
        v_ref[0, h] = v[:, sl]


def _attn_body(q_ref, k_ref, v_ref, ctx_ref, mean_ref):
    qi = pl.program_id(2)

    k = k_ref[0, 0]
    v = v_ref[0, 0]

    part = jnp.zeros((1, DH), jnp.float32)
    for half in range(NCH):
        q = q_ref[0, 0, half * HQ:(half + 1) * HQ]
        scores = lax.dot_general(q, k, _CONTRACT_LAST,
                                 preferred_element_type=jnp.float32)
        m = jnp.max(scores, axis=1, keepdims=True)
        p = jnp.exp2(scores - m)
        l = jnp.sum(p, axis=1, keepdims=True)
        ctx = lax.dot_general(p.astype(jnp.bfloat16), v, _CONTRACT_STD,
                              preferred_element_type=jnp.float32) / l
        ctx_ref[0, 0, half * HQ:(half + 1) * HQ] = ctx.astype(jnp.bfloat16)
        part = part + jnp.sum(ctx, axis=0, keepdims=True)

    @pl.when(qi == 0)
    def _():
        mean_ref[...] = jnp.zeros_like(mean_ref)

    mean_ref[0, 0, 0:1, :] += part * (1.0 / S)


def _gate_body(g_ref, wg_ref, bg_ref, be_ref, w_ref, bc_ref):
    g = g_ref[...]
    logits = lax.dot_general(g, wg_ref[...], _CONTRACT_LAST,
                             preferred_element_type=jnp.float32) + bg_ref[...]
    m = jnp.max(logits, axis=1, keepdims=True)
    pexp = jnp.exp(logits - m)
    probs = pexp / jnp.sum(pexp, axis=1, keepdims=True)

    eidx = lax.broadcasted_iota(jnp.int32, (B, E), 1)
    v1 = jnp.max(probs, axis=1, keepdims=True)
    i1 = jnp.min(jnp.where(probs >= v1, eidx, E), axis=1, keepdims=True)
    mask1 = eidx == i1
    p2 = jnp.where(mask1, -1.0, probs)
    v2 = jnp.max(p2, axis=1, keepdims=True)
    i2 = jnp.min(jnp.where(p2 >= v2, eidx, E), axis=1, keepdims=True)
    mask2 = eidx == i2

    w = jnp.where(mask1, v1, 0.0) + jnp.where(mask2, v2, 0.0)
    w_ref[...] = w
    bc_ref[...] = lax.dot_general(w, be_ref[...], _CONTRACT_STD,
                                  preferred_element_type=jnp.float32)


def _combine_body(w_ref, we_ref, wc_ref):
    we = we_ref[...].astype(jnp.float32)
    for b in range(B):
        acc = w_ref[b, 0] * we[0]
        for e in range(1, E):
            acc = acc + w_ref[b, e] * we[e]
        wc_ref[b] = acc.astype(jnp.bfloat16)


def _moe_out_body(ctx_ref, wc_ref, bc_ref, wo_ref, bo_ref, out_ref):
    ctx = jnp.concatenate([ctx_ref[0, h] for h in range(H)], axis=1)
    moe = lax.dot_general(ctx, wc_ref[0], _CONTRACT_LAST,
                          preferred_element_type=jnp.float32) + bc_ref[0]
    out = lax.dot_general(moe.astype(jnp.bfloat16), wo_ref[...], _CONTRACT_LAST,
                          preferred_element_type=jnp.float32) + bo_ref[...]
    out_ref[0] = out


def kernel(hidden_states, Wq, bq, Wk, bk, Wv, bv, We, be, Wg, bg, Wo, bo):
    scale = math.log2(math.e) / math.sqrt(DH)
    wq16 = (Wq * scale).astype(jnp.bfloat16)
    wk16 = Wk.astype(jnp.bfloat16)
    wv16 = Wv.astype(jnp.bfloat16)
    we16 = We.astype(jnp.bfloat16)
    wo16 = Wo.astype(jnp.bfloat16)
    bq2 = (bq * scale).reshape(1, D)
    bk2 = bk.reshape(1, D)
    bv2 = bv.reshape(1, D)
    bg2 = bg.reshape(1, E)
    bo2 = bo.reshape(1, D)

    n_sb = S // SBLK
    qkv_struct = jax.ShapeDtypeStruct((B, H, S, DH), jnp.bfloat16)
    q, k, v = pl.pallas_call(
        _qkv_body,
        grid=(B, n_sb),
        in_specs=[
            pl.BlockSpec((1, SBLK, D), lambda b, si: (b, si, 0)),
            pl.BlockSpec((D, D), lambda b, si: (0, 0)),
            pl.BlockSpec((1, D), lambda b, si: (0, 0)),
            pl.BlockSpec((D, D), lambda b, si: (0, 0)),
            pl.BlockSpec((1, D), lambda b, si: (0, 0)),
            pl.BlockSpec((D, D), lambda b, si: (0, 0)),
            pl.BlockSpec((1, D), lambda b, si: (0, 0)),
        ],
        out_specs=[
            pl.BlockSpec((1, H, SBLK, DH), lambda b, si: (b, 0, si, 0)),
            pl.BlockSpec((1, H, SBLK, DH), lambda b, si: (b, 0, si, 0)),
            pl.BlockSpec((1, H, SBLK, DH), lambda b, si: (b, 0, si, 0)),
        ],
        out_shape=[qkv_struct] * 3,
    )(hidden_states, wq16, bq2, wk16, bk2, wv16, bv2)

    nq = S // SQ
    ctx4, means = pl.pallas_call(
        _attn_body,
        grid=(B, H, nq),
        in_specs=[
            pl.BlockSpec((1, 1, SQ, DH), lambda b, h, qi: (b, h, qi, 0)),
            pl.BlockSpec((1, 1, S, DH), lambda b, h, qi: (b, h, 0, 0)),
            pl.BlockSpec((1, 1, S, DH), lambda b, h, qi: (b, h, 0, 0)),
        ],
        out_specs=[
            pl.BlockSpec((1, 1, SQ, DH), lambda b, h, qi: (b, h, qi, 0)),
            pl.BlockSpec((1, 1, 8, DH), lambda b, h, qi: (b, h, 0, 0)),
        ],
        out_shape=[
            jax.ShapeDtypeStruct((B, H, S, DH), jnp.bfloat16),
            jax.ShapeDtypeStruct((B, H, 8, DH), jnp.float32),
        ],
    )(q, k, v)

    gate_input = means[:, :, 0, :].reshape(B, D)

    w, bc = pl.pallas_call(
        _gate_body,
        grid=(1,),
        in_specs=[
            pl.BlockSpec((B, D), lambda i: (0, 0)),
            pl.BlockSpec((E, D), lambda i: (0, 0)),
            pl.BlockSpec((1, E), lambda i: (0, 0)),
            pl.BlockSpec((E, D), lambda i: (0, 0)),
        ],
        out_specs=[
            pl.BlockSpec((B, E), lambda i: (0, 0)),
            pl.BlockSpec((B, D), lambda i: (0, 0)),
        ],
        out_shape=[
            jax.ShapeDtypeStruct((B, E), jnp.float32),
            jax.ShapeDtypeStruct((B, D), jnp.float32),
        ],
    )(gate_input, Wg, bg2, be)

    n_rb = D // RB
    wc = pl.pallas_call(
        _combine_body,
        grid=(n_rb,),
        in_specs=[
            pl.BlockSpec(memory_space=pltpu.SMEM),
            pl.BlockSpec((E, RB, D), lambda i: (0, i, 0)),
        ],
        out_specs=pl.BlockSpec((B, RB, D), lambda i: (0, i, 0)),
        out_shape=jax.ShapeDtypeStruct((B, D, D), jnp.bfloat16),
    )(w, we16)

    bc3 = bc.reshape(B, 1, D)
    ns = S // SB
    out = pl.pallas_call(
        _moe_out_body,
        grid=(B, ns),
        in_specs=[
            pl.BlockSpec((1, H, SB, DH), lambda b, si: (b, 0, si, 0)),
            pl.BlockSpec((1, D, D), lambda b, si: (b, 0, 0)),
            pl.BlockSpec((1, 1, D), lambda b, si: (b, 0, 0)),
            pl.BlockSpec((D, D), lambda b, si: (0, 0)),
            pl.BlockSpec((1, D), lambda b, si: (0, 0)),
        ],
        out_specs=pl.BlockSpec((1, SB, D), lambda b, si: (b, si, 0)),
        out_shape=jax.ShapeDtypeStruct((B, S, D), jnp.float32),
    )(ctx4, wc, bc3, wo16, bo2)

    return out
